# Initial kernel scaffold; baseline (speedup 1.0000x reference)
import jax
import jax.numpy as jnp
from jax.experimental import pallas as pl


def kernel(node_attention, memorized_embedding, rel_emb, query_src_emb, query_rel_emb, query_time_emb, eg_idx, idx_vi, idx_vj, seg_softmax, dst_idx, proj_W, proj_b, left_W, left_b, right_W, right_b, center_W, center_b, lin_W, lin_b):
    raise NotImplementedError("write your pallas kernel here")


# speedup vs baseline: 1.1163x; 1.1163x over previous
import functools

import jax
import jax.numpy as jnp
from jax.experimental import pallas as pl

_N_MEM = 50000
_E = 320000
_B = 64
_D = 128
_DSM = 64
_NSEG = 40000
_NNEW = 40000

_EBLK = 4000
_NEB = _E // _EBLK
_MBLK = 1000


def _leaky(x):
    return jnp.where(x >= 0, x, 0.01 * x)


def _tables_body(mem_ref, ml_ref, mr_ref, lw_ref, lb_ref,
                 tl_ref, tr_ref, msg_ref):
    x = mem_ref[...]
    tl_ref[...] = jnp.dot(x, ml_ref[...], preferred_element_type=jnp.float32)
    tr_ref[...] = jnp.dot(x, mr_ref[...], preferred_element_type=jnp.float32)
    msg_ref[...] = (
        jnp.dot(x, lw_ref[...], preferred_element_type=jnp.float32)
        + lb_ref[...]
    )


def _make_tables(mem, ML, MR, lin_W, lin_b):
    return pl.pallas_call(
        _tables_body,
        grid=(_N_MEM // _MBLK,),
        in_specs=[
            pl.BlockSpec((_MBLK, _D), lambda i: (i, 0)),
            pl.BlockSpec((_D, _DSM), lambda i: (0, 0)),
            pl.BlockSpec((_D, _DSM), lambda i: (0, 0)),
            pl.BlockSpec((_D, _D), lambda i: (0, 0)),
            pl.BlockSpec((1, _D), lambda i: (0, 0)),
        ],
        out_specs=[
            pl.BlockSpec((_MBLK, _DSM), lambda i: (i, 0)),
            pl.BlockSpec((_MBLK, _DSM), lambda i: (i, 0)),
            pl.BlockSpec((_MBLK, _D), lambda i: (i, 0)),
        ],
        out_shape=[
            jax.ShapeDtypeStruct((_N_MEM, _DSM), jnp.float32),
            jax.ShapeDtypeStruct((_N_MEM, _DSM), jnp.float32),
            jax.ShapeDtypeStruct((_N_MEM, _D), jnp.float32),
        ],
    )(mem, ML, MR, lin_W, lin_b.reshape(1, _D))


def _logits_body(rel_ref, tlg_ref, trg_ref, eg_ref,
                 mrl_ref, mrr_ref, qlt_ref, qrt_ref,
                 bl_ref, br_ref, cw_ref, cb_ref,
                 out_ref):
    rel = rel_ref[...]
    eg = eg_ref[...]
    onehot = (eg.reshape(_EBLK, 1)
              == jax.lax.broadcasted_iota(jnp.int32, (1, _B), 1)
              ).astype(jnp.float32)
    qlg = jnp.dot(onehot, qlt_ref[...], preferred_element_type=jnp.float32)
    qrg = jnp.dot(onehot, qrt_ref[...], preferred_element_type=jnp.float32)
    l_pre = (tlg_ref[...] + qlg + bl_ref[...]
             + jnp.dot(rel, mrl_ref[...], preferred_element_type=jnp.float32))
    r_pre = (trg_ref[...] + qrg + br_ref[...]
             + jnp.dot(rel, mrr_ref[...], preferred_element_type=jnp.float32))
    left_h = _leaky(l_pre)
    right_h = (jnp.dot(_leaky(r_pre), cw_ref[...],
                       preferred_element_type=jnp.float32) + cb_ref[...])
    out_ref[...] = jnp.sum(left_h * right_h, axis=1).reshape(1, 1, _EBLK)


def _make_logits(rel_emb, TLg, TRg, eg2d, MrelL, MrelR, QLt, QRt,
                 bias_l, bias_r, center_W, center_b):
    out = pl.pallas_call(
        _logits_body,
        grid=(_NEB,),
        in_specs=[
            pl.BlockSpec((_EBLK, _D), lambda i: (i, 0)),
            pl.BlockSpec((_EBLK, _DSM), lambda i: (i, 0)),
            pl.BlockSpec((_EBLK, _DSM), lambda i: (i, 0)),
            pl.BlockSpec((1, 1, _EBLK), lambda i: (i, 0, 0)),
            pl.BlockSpec((_D, _DSM), lambda i: (0, 0)),
            pl.BlockSpec((_D, _DSM), lambda i: (0, 0)),
            pl.BlockSpec((_B, _DSM), lambda i: (0, 0)),
            pl.BlockSpec((_B, _DSM), lambda i: (0, 0)),
            pl.BlockSpec((1, _DSM), lambda i: (0, 0)),
            pl.BlockSpec((1, _DSM), lambda i: (0, 0)),
            pl.BlockSpec((_DSM, _DSM), lambda i: (0, 0)),
            pl.BlockSpec((1, _DSM), lambda i: (0, 0)),
        ],
        out_specs=pl.BlockSpec((1, 1, _EBLK), lambda i: (i, 0, 0)),
        out_shape=jax.ShapeDtypeStruct((_NEB, 1, _EBLK), jnp.float32),
    )(rel_emb, TLg, TRg, eg2d, MrelL, MrelR, QLt, QRt,
      bias_l.reshape(1, _DSM), bias_r.reshape(1, _DSM),
      center_W, center_b.reshape(1, _DSM))
    return out.reshape(_E)


def kernel(node_attention, memorized_embedding, rel_emb, query_src_emb,
           query_rel_emb, query_time_emb, eg_idx, idx_vi, idx_vj,
           seg_softmax, dst_idx, proj_W, proj_b, left_W, left_b,
           right_W, right_b, center_W, center_b, lin_W, lin_b):
    f32 = jnp.float32
    L1, L2, L3, L4, L5 = (left_W[k * _DSM:(k + 1) * _DSM] for k in range(5))
    R1, R2, R3, R4, R5 = (right_W[k * _DSM:(k + 1) * _DSM] for k in range(5))
    ML = proj_W @ L1
    MR = proj_W @ R1
    MrelL = proj_W @ L2
    MrelR = proj_W @ R2
    qs = query_src_emb @ proj_W + proj_b
    qr = query_rel_emb @ proj_W + proj_b
    qt = query_time_emb @ proj_W + proj_b
    QLt = qs @ L3 + qr @ L4 + qt @ L5
    QRt = qs @ R3 + qr @ R4 + qt @ R5
    bias_l = left_b + proj_b @ L1 + proj_b @ L2
    bias_r = right_b + proj_b @ R1 + proj_b @ R2

    TL, TR, MSG = _make_tables(memorized_embedding, ML, MR, lin_W, lin_b)

    TLg = jnp.take(TL, idx_vi, axis=0)
    TRg = jnp.take(TR, idx_vj, axis=0)
    MSGg = jnp.take(MSG, idx_vj, axis=0)

    eg2d = eg_idx.reshape(_NEB, 1, _EBLK)
    logits = _make_logits(rel_emb, TLg, TRg, eg2d, MrelL, MrelR, QLt, QRt,
                          bias_l, bias_r, center_W, center_b)

    seg_max = jax.ops.segment_max(logits, seg_softmax, num_segments=_NSEG)
    ex = jnp.exp(logits - seg_max[seg_softmax])
    seg_sum = jax.ops.segment_sum(ex, seg_softmax, num_segments=_NSEG)
    transition = ex / (seg_sum[seg_softmax] + 1e-16)

    new_att = jax.ops.segment_sum(transition * node_attention, dst_idx,
                                  num_segments=_NNEW)
    upd = jax.ops.segment_sum(transition[:, None] * MSGg, dst_idx,
                              num_segments=_NNEW)
    return (new_att, upd)


# speedup vs baseline: 2.0498x; 1.8363x over previous
import functools

import jax
import jax.numpy as jnp
from jax.experimental import pallas as pl
from jax.experimental.pallas import tpu as pltpu

_N_MEM = 50000
_E = 320000
_B = 64
_D = 128
_DSM = 64
_NSEG = 40000
_NNEW = 40000

_EBLK = 4000
_NEB = _E // _EBLK
_MBLK = 1000


def _leaky(x):
    return jnp.where(x >= 0, x, 0.01 * x)


def _tables_body(mem_ref, ml_ref, mr_ref, lw_ref, lb_ref,
                 tl_ref, tr_ref, msg_ref):
    x = mem_ref[...]
    tl_ref[...] = jnp.dot(x, ml_ref[...], preferred_element_type=jnp.float32)
    tr_ref[...] = jnp.dot(x, mr_ref[...], preferred_element_type=jnp.float32)
    msg_ref[...] = (
        jnp.dot(x, lw_ref[...], preferred_element_type=jnp.float32)
        + lb_ref[...]
    )


def _make_tables(mem, ML, MR, lin_W, lin_b):
    return pl.pallas_call(
        _tables_body,
        grid=(_N_MEM // _MBLK,),
        in_specs=[
            pl.BlockSpec((_MBLK, _D), lambda i: (i, 0)),
            pl.BlockSpec((_D, _DSM), lambda i: (0, 0)),
            pl.BlockSpec((_D, _DSM), lambda i: (0, 0)),
            pl.BlockSpec((_D, _D), lambda i: (0, 0)),
            pl.BlockSpec((1, _D), lambda i: (0, 0)),
        ],
        out_specs=[
            pl.BlockSpec((_MBLK, _DSM), lambda i: (i, 0)),
            pl.BlockSpec((_MBLK, _DSM), lambda i: (i, 0)),
            pl.BlockSpec((_MBLK, _D), lambda i: (i, 0)),
        ],
        out_shape=[
            jax.ShapeDtypeStruct((_N_MEM, _DSM), jnp.float32),
            jax.ShapeDtypeStruct((_N_MEM, _DSM), jnp.float32),
            jax.ShapeDtypeStruct((_N_MEM, _D), jnp.float32),
        ],
    )(mem, ML, MR, lin_W, lin_b.reshape(1, _D))


def _logits_body(rel_ref, tlg_ref, trg_ref, eg_ref,
                 mrl_ref, mrr_ref, qlt_ref, qrt_ref,
                 bl_ref, br_ref, cw_ref, cb_ref,
                 out_ref):
    rel = rel_ref[...]
    eg = eg_ref[...]
    onehot = (eg.reshape(_EBLK, 1)
              == jax.lax.broadcasted_iota(jnp.int32, (1, _B), 1)
              ).astype(jnp.float32)
    qlg = jnp.dot(onehot, qlt_ref[...], preferred_element_type=jnp.float32)
    qrg = jnp.dot(onehot, qrt_ref[...], preferred_element_type=jnp.float32)
    l_pre = (tlg_ref[...] + qlg + bl_ref[...]
             + jnp.dot(rel, mrl_ref[...], preferred_element_type=jnp.float32))
    r_pre = (trg_ref[...] + qrg + br_ref[...]
             + jnp.dot(rel, mrr_ref[...], preferred_element_type=jnp.float32))
    left_h = _leaky(l_pre)
    right_h = (jnp.dot(_leaky(r_pre), cw_ref[...],
                       preferred_element_type=jnp.float32) + cb_ref[...])
    out_ref[...] = jnp.sum(left_h * right_h, axis=1).reshape(1, 1, _EBLK)


def _make_logits(rel_emb, TLg, TRg, eg2d, MrelL, MrelR, QLt, QRt,
                 bias_l, bias_r, center_W, center_b):
    out = pl.pallas_call(
        _logits_body,
        grid=(_NEB,),
        in_specs=[
            pl.BlockSpec((_EBLK, _D), lambda i: (i, 0)),
            pl.BlockSpec((_EBLK, _DSM), lambda i: (i, 0)),
            pl.BlockSpec((_EBLK, _DSM), lambda i: (i, 0)),
            pl.BlockSpec((1, 1, _EBLK), lambda i: (i, 0, 0)),
            pl.BlockSpec((_D, _DSM), lambda i: (0, 0)),
            pl.BlockSpec((_D, _DSM), lambda i: (0, 0)),
            pl.BlockSpec((_B, _DSM), lambda i: (0, 0)),
            pl.BlockSpec((_B, _DSM), lambda i: (0, 0)),
            pl.BlockSpec((1, _DSM), lambda i: (0, 0)),
            pl.BlockSpec((1, _DSM), lambda i: (0, 0)),
            pl.BlockSpec((_DSM, _DSM), lambda i: (0, 0)),
            pl.BlockSpec((1, _DSM), lambda i: (0, 0)),
        ],
        out_specs=pl.BlockSpec((1, 1, _EBLK), lambda i: (i, 0, 0)),
        out_shape=jax.ShapeDtypeStruct((_NEB, 1, _EBLK), jnp.float32),
    )(rel_emb, TLg, TRg, eg2d, MrelL, MrelR, QLt, QRt,
      bias_l.reshape(1, _DSM), bias_r.reshape(1, _DSM),
      center_W, center_b.reshape(1, _DSM))
    return out


def _shr(x, d, fill):
    return jnp.concatenate(
        [jnp.full((1, d), fill, x.dtype), x[:, :-d]], axis=1)


def _shl(x, d, fill):
    return jnp.concatenate(
        [x[:, d:], jnp.full((1, d), fill, x.dtype)], axis=1)


def _lane_scalar(x, lane):
    sel = jax.lax.broadcasted_iota(jnp.int32, x.shape, 1) == lane
    return jnp.sum(jnp.where(sel, x, jnp.zeros_like(x)))


def _fwd_body(lg_ref, seg_ref, ex_ref, f_ref, cseg_ref, cc_ref, cf_ref):
    pid = pl.program_id(0)

    @pl.when(pid == 0)
    def _():
        cseg_ref[0] = -1
        cc_ref[0] = 0.0
        cf_ref[0] = 0.0

    l = lg_ref[...].reshape(1, _EBLK)
    s = seg_ref[...].reshape(1, _EBLK)
    f0 = (s != _shr(s, 1, -1)).astype(jnp.int32)
    v, fb = l, f0
    d = 1
    while d < _EBLK:
        v = jnp.where(fb > 0, v, _shr(v, d, 0.0))
        fb = jnp.maximum(fb, _shr(fb, d, 1))
        d *= 2
    mask = s == cseg_ref[0]
    c = jnp.where(mask, cc_ref[0], v)
    ex = jnp.exp(l - c)
    v2, fb2 = ex, f0
    d = 1
    while d < _EBLK:
        v2 = v2 + jnp.where(fb2 > 0, 0.0, _shr(v2, d, 0.0))
        fb2 = jnp.maximum(fb2, _shr(fb2, d, 1))
        d *= 2
    F = v2 + jnp.where(mask, cf_ref[0], 0.0)
    ex_ref[...] = ex.reshape(1, 1, _EBLK)
    f_ref[...] = F.reshape(1, 1, _EBLK)
    cseg_ref[0] = _lane_scalar(s, _EBLK - 1)
    cc_ref[0] = _lane_scalar(c, _EBLK - 1)
    cf_ref[0] = _lane_scalar(F, _EBLK - 1)


def _bwd_body(ex_ref, f_ref, seg_ref, natt_ref, t_ref, w_ref,
              cseg_ref, ct_ref):
    pid = pl.program_id(0)

    @pl.when(pid == 0)
    def _():
        cseg_ref[0] = -1
        ct_ref[0] = 0.0

    ex = ex_ref[...].reshape(1, _EBLK)
    F = f_ref[...].reshape(1, _EBLK)
    s = seg_ref[...].reshape(1, _EBLK)
    e0 = (s != _shl(s, 1, -1)).astype(jnp.int32)
    v, fb = F, e0
    d = 1
    while d < _EBLK:
        v = jnp.where(fb > 0, v, _shl(v, d, 0.0))
        fb = jnp.maximum(fb, _shl(fb, d, 1))
        d *= 2
    mask = s == cseg_ref[0]
    T = jnp.where(mask, ct_ref[0], v)
    t = ex / (T + 1e-16)
    w = t * natt_ref[...].reshape(1, _EBLK)
    t_ref[...] = t.reshape(1, 1, _EBLK)
    w_ref[...] = w.reshape(1, 1, _EBLK)
    cseg_ref[0] = _lane_scalar(s, 0)
    ct_ref[0] = _lane_scalar(T, 0)


def _seg_softmax(logits3, seg3, natt3):
    spec = pl.BlockSpec((1, 1, _EBLK), lambda i: (i, 0, 0))
    ex, F = pl.pallas_call(
        _fwd_body,
        grid=(_NEB,),
        in_specs=[spec, spec],
        out_specs=[spec, spec],
        out_shape=[jax.ShapeDtypeStruct((_NEB, 1, _EBLK), jnp.float32),
                   jax.ShapeDtypeStruct((_NEB, 1, _EBLK), jnp.float32)],
        scratch_shapes=[pltpu.SMEM((1,), jnp.int32),
                        pltpu.SMEM((1,), jnp.float32),
                        pltpu.SMEM((1,), jnp.float32)],
    )(logits3, seg3)
    rspec = pl.BlockSpec((1, 1, _EBLK), lambda i: (_NEB - 1 - i, 0, 0))
    t, w = pl.pallas_call(
        _bwd_body,
        grid=(_NEB,),
        in_specs=[rspec, rspec, rspec, rspec],
        out_specs=[rspec, rspec],
        out_shape=[jax.ShapeDtypeStruct((_NEB, 1, _EBLK), jnp.float32),
                   jax.ShapeDtypeStruct((_NEB, 1, _EBLK), jnp.float32)],
        scratch_shapes=[pltpu.SMEM((1,), jnp.int32),
                        pltpu.SMEM((1,), jnp.float32)],
    )(ex, F, seg3, natt3)
    return t, w


def kernel(node_attention, memorized_embedding, rel_emb, query_src_emb,
           query_rel_emb, query_time_emb, eg_idx, idx_vi, idx_vj,
           seg_softmax, dst_idx, proj_W, proj_b, left_W, left_b,
           right_W, right_b, center_W, center_b, lin_W, lin_b):
    f32 = jnp.float32
    L1, L2, L3, L4, L5 = (left_W[k * _DSM:(k + 1) * _DSM] for k in range(5))
    R1, R2, R3, R4, R5 = (right_W[k * _DSM:(k + 1) * _DSM] for k in range(5))
    ML = proj_W @ L1
    MR = proj_W @ R1
    MrelL = proj_W @ L2
    MrelR = proj_W @ R2
    qs = query_src_emb @ proj_W + proj_b
    qr = query_rel_emb @ proj_W + proj_b
    qt = query_time_emb @ proj_W + proj_b
    QLt = qs @ L3 + qr @ L4 + qt @ L5
    QRt = qs @ R3 + qr @ R4 + qt @ R5
    bias_l = left_b + proj_b @ L1 + proj_b @ L2
    bias_r = right_b + proj_b @ R1 + proj_b @ R2

    TL, TR, MSG = _make_tables(memorized_embedding, ML, MR, lin_W, lin_b)

    TLg = jnp.take(TL, idx_vi, axis=0)
    TRg = jnp.take(TR, idx_vj, axis=0)
    MSGg = jnp.take(MSG, idx_vj, axis=0)

    eg2d = eg_idx.reshape(_NEB, 1, _EBLK)
    logits3 = _make_logits(rel_emb, TLg, TRg, eg2d, MrelL, MrelR, QLt, QRt,
                           bias_l, bias_r, center_W, center_b)

    seg3 = seg_softmax.reshape(_NEB, 1, _EBLK)
    natt3 = node_attention.reshape(_NEB, 1, _EBLK)
    t3, w3 = _seg_softmax(logits3, seg3, natt3)
    transition = t3.reshape(_E)
    w = w3.reshape(_E)

    new_att = jax.ops.segment_sum(w, dst_idx, num_segments=_NNEW)
    upd = jax.ops.segment_sum(transition[:, None] * MSGg, dst_idx,
                              num_segments=_NNEW)
    return (new_att, upd)
